# Initial kernel scaffold; baseline (speedup 1.0000x reference)
import jax
import jax.numpy as jnp
from jax.experimental import pallas as pl


def kernel(x, edge_index, batch, Ws1, bs1, Ws2, bs2, gammas, betas):
    raise NotImplementedError("write your pallas kernel here")


# speedup vs baseline: 4.6725x; 4.6725x over previous
import functools

import jax
import jax.numpy as jnp
from jax import lax
from jax.experimental import pallas as pl
from jax.experimental.pallas import tpu as pltpu
from jax.experimental.pallas import tpu_sc as plsc

N_NODES = 10000
N_EDGES = 320000
D = 128
N_GRAPHS = 64
N_LAYERS = 3

NC = 2
NS = 16
NW = NC * NS
EDGES_PER_W = N_EDGES // NW
K = 80
N_CHUNKS = EDGES_PER_W // K
ROWS_PER_S = 624
ROW_TAIL = N_NODES - ROWS_PER_S * NS


def _seg_sum_body(h_hbm, src_hbm, dst_hbm, out_hbm, src_v, dst_v, rows_v, acc, sem):
    c = lax.axis_index("c")
    s = lax.axis_index("s")
    wid = s * NC + c
    r0 = s * ROWS_PER_S
    pltpu.sync_copy(h_hbm.at[pl.ds(r0, ROWS_PER_S)], acc.at[pl.ds(r0, ROWS_PER_S)])

    @pl.when(s == 0)
    def _():
        pltpu.sync_copy(h_hbm.at[pl.ds(ROWS_PER_S * NS, ROW_TAIL)],
                        acc.at[pl.ds(ROWS_PER_S * NS, ROW_TAIL)])

    plsc.subcore_barrier()

    base = wid * EDGES_PER_W

    def body(i, carry):
        off = base + i * K
        pltpu.sync_copy(src_hbm.at[pl.ds(off, K)], src_v)
        pltpu.sync_copy(dst_hbm.at[pl.ds(off, K)], dst_v)
        pltpu.async_copy(h_hbm.at[src_v], rows_v, sem).wait()
        pltpu.sync_copy(rows_v, acc.at[dst_v], add=True)
        return carry

    lax.fori_loop(0, N_CHUNKS, body, 0)
    plsc.subcore_barrier()
    pltpu.sync_copy(acc.at[pl.ds(r0, ROWS_PER_S)],
                    out_hbm.at[c].at[pl.ds(r0, ROWS_PER_S)])

    @pl.when(s == 0)
    def _():
        pltpu.sync_copy(acc.at[pl.ds(ROWS_PER_S * NS, ROW_TAIL)],
                        out_hbm.at[c].at[pl.ds(ROWS_PER_S * NS, ROW_TAIL)])


@functools.cache
def _make_seg_sum():
    return pl.kernel(
        _seg_sum_body,
        out_type=jax.ShapeDtypeStruct((NC, N_NODES, D), jnp.float32),
        mesh=plsc.VectorSubcoreMesh(core_axis_name="c", subcore_axis_name="s",
                                    num_cores=NC, num_subcores=NS),
        scratch_types=[
            pltpu.VMEM((K,), jnp.int32),
            pltpu.VMEM((K,), jnp.int32),
            pltpu.VMEM((K, D), jnp.float32),
            pltpu.VMEM_SHARED((N_NODES, D), jnp.float32),
            pltpu.SemaphoreType.DMA,
        ],
    )


def _mlp_body(h_ref, a_ref, batch_ref, w1_ref, b1_ref, w2_ref, b2_ref,
              g_ref, be_ref, z_ref, pool_ref):
    zin = a_ref[0] + a_ref[1] - h_ref[...]
    t = lax.dot_general(zin, w1_ref[...], (((1,), (1,)), ((), ())),
                        preferred_element_type=jnp.float32) + b1_ref[...]
    t = jnp.maximum(t, 0.0)
    t = lax.dot_general(t, w2_ref[...], (((1,), (1,)), ((), ())),
                        preferred_element_type=jnp.float32) + b2_ref[...]
    t = jnp.maximum(t, 0.0)
    mean = jnp.mean(t, axis=0, keepdims=True)
    var = jnp.mean((t - mean) ** 2, axis=0, keepdims=True)
    zo = (t - mean) * lax.rsqrt(var + 1e-5) * g_ref[...] + be_ref[...]
    z_ref[...] = zo
    onehot = (lax.broadcasted_iota(jnp.int32, (N_GRAPHS, N_NODES), 0)
              == batch_ref[...]).astype(jnp.float32)
    pool_ref[...] = lax.dot_general(onehot, zo, (((1,), (0,)), ((), ())),
                                    precision=lax.Precision.HIGHEST,
                                    preferred_element_type=jnp.float32)


_mlp = pl.pallas_call(
    _mlp_body,
    out_shape=(jax.ShapeDtypeStruct((N_NODES, D), jnp.float32),
               jax.ShapeDtypeStruct((N_GRAPHS, D), jnp.float32)),
)


def kernel(x, edge_index, batch, Ws1, bs1, Ws2, bs2, gammas, betas):
    src = edge_index[0].astype(jnp.int32)
    dst = edge_index[1].astype(jnp.int32)
    batch2d = batch.astype(jnp.int32).reshape(1, N_NODES)
    h = x
    pools = []
    for i in range(N_LAYERS):
        acc = _make_seg_sum()(h, src, dst)
        h, pool = _mlp(h, acc, batch2d,
                       Ws1[i], bs1[i].reshape(1, D),
                       Ws2[i], bs2[i].reshape(1, D),
                       gammas[i].reshape(1, D), betas[i].reshape(1, D))
        pools.append(pool)
    return jnp.concatenate(pools, axis=1)
